# baseline (speedup 1.0000x reference)
import functools

import jax
import jax.numpy as jnp
from jax import lax
from jax.experimental import pallas as pl
from jax.experimental.pallas import tpu as pltpu
from jax.experimental.pallas import tpu_sc as plsc

B = 16384
V = 1000000
D = 64
DOUT = D + 4
NC = 2
NS = 16
NW = NC * NS
L = 16

BLK = 128
BPT = 245
IDW = BPT * BLK
TAIL = (V // BLK) * BLK
CAP = 8192
ROUNDS = 2
NSLOT = 6
SROWS = 128
FT = SROWS - L
PIECE = 2048
BTRASH = B
OUTR = B + 64


def _iota():
    return lax.iota(jnp.int32, L)


def _body(idx_hbm, ffeat_hbm, stats_hbm, tt_hbm, ttail_hbm, out_hbm,
          ids_l, pos_l, pval_s, hist, starts, cursor, win, stage, spos,
          fbuf, idxp, stats_v,
          wsem0, wsem1, wsem2, wsem3, wsem4, wsem5, fsem, ssem):
    wid = lax.axis_index("s") * NC + lax.axis_index("c")
    lo = wid * IDW
    hi = lo + IDW
    wsems = (wsem0, wsem1, wsem2, wsem3, wsem4, wsem5)

    pltpu.sync_copy(stats_hbm, stats_v)

    def reset_spos():
        for rv in range(SROWS // L):
            spos[0, pl.ds(rv * L, L)] = jnp.full((L,), BTRASH, jnp.int32)

    reset_spos()

    def rs_of(c):
        return lo + c * BLK

    def fire(c, slot):
        rs = rs_of(c)
        ok = c < BPT
        @pl.when(ok & (rs < TAIL))
        def _():
            pltpu.async_copy(
                tt_hbm.at[:, pl.ds(pl.multiple_of(rs, BLK), BLK)],
                win.at[slot], wsems[slot])
        @pl.when(ok & (rs == TAIL))
        def _():
            pltpu.async_copy(ttail_hbm, win.at[slot], wsems[slot])

    def drain(c, slot):
        rs = rs_of(c)
        ok = c < BPT
        @pl.when(ok & (rs <= TAIL))
        def _():
            pltpu.make_async_copy(
                tt_hbm.at[:, pl.ds(0, BLK)], win.at[slot],
                wsems[slot]).wait()

    def flush():
        pltpu.async_copy(ffeat_hbm.at[spos.at[0]], fbuf, fsem).wait()
        for i in range(4):
            m = plsc.load_gather(stats_v, [jnp.full((L,), 1 + i, jnp.int32)])
            s = plsc.load_gather(stats_v, [jnp.full((L,), 5 + i, jnp.int32)])
            col = jnp.full((L,), D + i, jnp.int32)
            fcol = jnp.full((L,), i, jnp.int32)
            for rv in range(SROWS // L):
                rows = _iota() + rv * L
                x = plsc.load_gather(fbuf, [rows, fcol])
                plsc.store_scatter(stage, [rows, col], (x - m) * s)
        pltpu.async_copy(stage, out_hbm.at[spos.at[0]], ssem).wait()
        reset_spos()

    def total_scan(skip):
        def piece(p, g):
            pltpu.sync_copy(
                idx_hbm.at[pl.ds(pl.multiple_of(p * PIECE, PIECE), PIECE)],
                idxp)
            def vreg(v, gv):
                ids = idxp[pl.ds(v * L, L)]
                pos = _iota() + p * PIECE + v * L
                m = (ids >= lo) & (ids < hi)
                pc = lax.cumsum(m.astype(jnp.int32))
                gidx = gv + pc - 1
                keep = m & (gidx >= skip) & (gidx < skip + CAP)
                slot = gidx - skip
                plsc.store_scatter(ids_l, [slot], ids, mask=keep)
                plsc.store_scatter(pos_l, [slot], pos, mask=keep)
                return gv + jnp.sum(m.astype(jnp.int32))
            return lax.fori_loop(0, PIECE // L, vreg, g)
        return lax.fori_loop(0, B // PIECE, piece, jnp.int32(0))

    def extract(tbl, c):
        v = tbl[pl.ds(pl.multiple_of((c >> 4) * L, L), L)]
        return jnp.sum(jnp.where(_iota() == (c & (L - 1)), v, 0))

    def do_round(r, total0):
        skip = r * CAP
        total = lax.cond(r == 0,
                         lambda _: total_scan(skip),
                         lambda t: lax.cond(t > skip,
                                            lambda tt: total_scan(skip),
                                            lambda tt: tt,
                                            t),
                         total0)
        n = jnp.clip(total - skip, 0, CAP)
        nv = (n + L - 1) // L

        @pl.when(n > 0)
        def _():
            for b in range(256 // L):
                hist[pl.ds(b * L, L)] = jnp.zeros((L,), jnp.int32)

            def histp(v, _):
                ids = ids_l[pl.ds(v * L, L)]
                lanes = _iota() + v * L
                valid = lanes < n
                ch = jnp.where(valid, (ids - lo) >> 7, 255)
                plsc.addupdate_scatter(hist, [ch],
                                       jnp.ones((L,), jnp.int32),
                                       mask=valid)
                return 0
            lax.fori_loop(0, nv, histp, 0)

            carry = jnp.int32(0)
            for b in range(256 // L):
                h = hist[pl.ds(b * L, L)]
                excl = carry + lax.cumsum(h) - h
                starts[pl.ds(b * L, L)] = excl
                cursor[pl.ds(b * L, L)] = excl
                carry = carry + jnp.sum(h)

            def place(v, _):
                ids = ids_l[pl.ds(v * L, L)]
                pos = pos_l[pl.ds(v * L, L)]
                lanes = _iota() + v * L
                valid = lanes < n
                ch = jnp.where(valid, (ids - lo) >> 7, 255)
                pval = (ch << 21) | ((ids & (BLK - 1)) << 14) | pos
                chs, pvs = plsc.sort_key_val(ch, pval)
                vs = chs < 255
                rank = plsc.scan_count(chs)[0] - 1
                base = plsc.load_gather(cursor, [chs])
                plsc.store_scatter(pval_s, [base + rank], pvs, mask=vs)
                plsc.addupdate_scatter(cursor, [chs],
                                       jnp.ones((L,), jnp.int32),
                                       mask=vs)
                return 0
            lax.fori_loop(0, nv, place, 0)

            for s in range(NSLOT):
                fire(jnp.int32(s), s)

            def wave(wv, sn_w):
                c0 = wv * NSLOT
                for s in range(NSLOT):
                    drain(c0 + s, s)
                st = extract(starts, c0)
                en = extract(starts, c0 + NSLOT)
                cnt = en - st

                def itemg(g, sn_g):
                    addr = st + g * L + _iota()
                    m = addr < en
                    pv = plsc.load_gather(pval_s, [addr], mask=m)
                    slotv = (pv >> 21) - c0
                    loc = (pv >> 14) & (BLK - 1)
                    pos = pv & ((1 << 14) - 1)
                    slot = sn_g + lax.cumsum(m.astype(jnp.int32)) - 1
                    for j in range(D):
                        vals = plsc.load_gather(
                            win, [slotv, jnp.full((L,), j, jnp.int32), loc],
                            mask=m)
                        plsc.store_scatter(
                            stage, [slot, jnp.full((L,), j, jnp.int32)],
                            vals, mask=m)
                    plsc.store_scatter(
                        spos, [jnp.zeros((L,), jnp.int32), slot],
                        pos, mask=m)
                    sn2 = sn_g + jnp.sum(m.astype(jnp.int32))

                    def doflush(x):
                        flush()
                        return jnp.int32(0)

                    return lax.cond(sn2 >= FT, doflush, lambda x: x, sn2)

                sn_w = lax.fori_loop(0, (cnt + L - 1) // L, itemg, sn_w)
                for s in range(NSLOT):
                    fire(c0 + NSLOT + s, s)
                return sn_w

            snf = lax.fori_loop(0, (BPT + NSLOT - 1) // NSLOT, wave,
                                jnp.int32(0))

            @pl.when(snf > 0)
            def _():
                flush()

        return total

        return 0

    lax.fori_loop(0, ROUNDS, do_round, jnp.int32(0))


def _sc_call(idx, ffeat, stats, tt, ttail):
    mesh = plsc.VectorSubcoreMesh(core_axis_name="c", subcore_axis_name="s")
    run = functools.partial(
        pl.kernel,
        mesh=mesh,
        compiler_params=pltpu.CompilerParams(use_tc_tiling_on_sc=True,
                                             needs_layout_passes=False),
        out_type=jax.ShapeDtypeStruct((OUTR, 128), jnp.float32),
        scratch_types=[
            pltpu.VMEM((CAP,), jnp.int32),
            pltpu.VMEM((CAP,), jnp.int32),
            pltpu.VMEM((CAP,), jnp.int32),
            pltpu.VMEM((256,), jnp.int32),
            pltpu.VMEM((256,), jnp.int32),
            pltpu.VMEM((256,), jnp.int32),
            pltpu.VMEM((NSLOT, D, BLK), jnp.float32),
            pltpu.VMEM((SROWS, 128), jnp.float32),
            pltpu.VMEM((1, SROWS), jnp.int32),
            pltpu.VMEM((SROWS, 128), jnp.float32),
            pltpu.VMEM((PIECE,), jnp.int32),
            pltpu.VMEM((L,), jnp.float32),
            pltpu.SemaphoreType.DMA,
            pltpu.SemaphoreType.DMA,
            pltpu.SemaphoreType.DMA,
            pltpu.SemaphoreType.DMA,
            pltpu.SemaphoreType.DMA,
            pltpu.SemaphoreType.DMA,
            pltpu.SemaphoreType.DMA,
            pltpu.SemaphoreType.DMA,
        ],
    )(_body)
    return run(idx, ffeat, stats, tt, ttail)


def kernel(visitorid, user_number_of_views, user_number_of_addtocart,
           user_number_of_purchases, number_of_unique_items,
           table, norm_mean, norm_var):
    idx = visitorid.astype(jnp.int32)
    inv_std = lax.rsqrt(norm_var.astype(jnp.float32) + 1e-7)
    stats = jnp.concatenate(
        [jnp.zeros((1,), jnp.float32), norm_mean.astype(jnp.float32),
         inv_std, jnp.zeros((L - 9,), jnp.float32)])
    feats = jnp.stack(
        [user_number_of_views, user_number_of_addtocart,
         user_number_of_purchases, number_of_unique_items], axis=1)
    ffeat = jnp.zeros((OUTR, 128), jnp.float32).at[:B, :4].set(feats)
    tt = table.T
    ttail = jnp.zeros((D, 128), jnp.float32).at[:, :V - TAIL].set(
        table[TAIL:].T)
    out = _sc_call(idx, ffeat, stats, tt, ttail)
    return out[:B, :DOUT]


# speedup vs baseline: 1.0205x; 1.0205x over previous
import functools

import jax
import jax.numpy as jnp
from jax import lax
from jax.experimental import pallas as pl
from jax.experimental.pallas import tpu as pltpu
from jax.experimental.pallas import tpu_sc as plsc

B = 16384
V = 1000000
D = 64
DOUT = D + 4
NC = 2
NS = 16
NW = NC * NS
L = 16

BLK = 128
BPT = 245
IDW = BPT * BLK
TAIL = (V // BLK) * BLK
CAP = 8192
ROUNDS = 2
NSLOT = 6
SROWS = 128
FT = SROWS - L
PIECE = 2048
BTRASH = B
OUTR = B + 64


def _iota():
    return lax.iota(jnp.int32, L)


def _body(idx_hbm, ffeat_hbm, stats_hbm, tt_hbm, ttail_hbm, out_hbm,
          ids_l, pos_l, pval_s, hist, starts, cursor, win, stage, spos,
          fbuf, idxp, stats_v,
          wsem0, wsem1, wsem2, wsem3, wsem4, wsem5, fsem, ssem):
    wid = lax.axis_index("s") * NC + lax.axis_index("c")
    lo = wid * IDW
    hi = lo + IDW
    wsems = (wsem0, wsem1, wsem2, wsem3, wsem4, wsem5)

    pltpu.sync_copy(stats_hbm, stats_v)

    def reset_spos():
        for rv in range(SROWS // L):
            spos[0, pl.ds(rv * L, L)] = jnp.full((L,), BTRASH, jnp.int32)

    reset_spos()

    def rs_of(c):
        return lo + c * BLK

    def fire(c, slot):
        rs = rs_of(c)
        ok = c < BPT
        @pl.when(ok & (rs < TAIL))
        def _():
            pltpu.async_copy(
                tt_hbm.at[:, pl.ds(pl.multiple_of(rs, BLK), BLK)],
                win.at[slot], wsems[slot])
        @pl.when(ok & (rs == TAIL))
        def _():
            pltpu.async_copy(ttail_hbm, win.at[slot], wsems[slot])

    def drain(c, slot):
        rs = rs_of(c)
        ok = c < BPT
        @pl.when(ok & (rs <= TAIL))
        def _():
            pltpu.make_async_copy(
                tt_hbm.at[:, pl.ds(0, BLK)], win.at[slot],
                wsems[slot]).wait()

    def flush():
        pltpu.async_copy(ffeat_hbm.at[spos.at[0]], fbuf, fsem).wait()
        for i in range(4):
            m = plsc.load_gather(stats_v, [jnp.full((L,), 1 + i, jnp.int32)])
            s = plsc.load_gather(stats_v, [jnp.full((L,), 5 + i, jnp.int32)])
            col = jnp.full((L,), D + i, jnp.int32)
            fcol = jnp.full((L,), i, jnp.int32)
            for rv in range(SROWS // L):
                rows = _iota() + rv * L
                x = plsc.load_gather(fbuf, [rows, fcol])
                plsc.store_scatter(stage, [rows, col], (x - m) * s)
        pltpu.async_copy(stage, out_hbm.at[spos.at[0]], ssem).wait()
        reset_spos()

    def total_scan(skip):
        def piece(p, g):
            pltpu.sync_copy(
                idx_hbm.at[pl.ds(pl.multiple_of(p * PIECE, PIECE), PIECE)],
                idxp)
            def vreg(v, gv):
                ids = idxp[pl.ds(v * L, L)]
                pos = _iota() + p * PIECE + v * L
                m = (ids >= lo) & (ids < hi)
                pc = lax.cumsum(m.astype(jnp.int32))
                gidx = gv + pc - 1
                keep = m & (gidx >= skip) & (gidx < skip + CAP)
                slot = gidx - skip
                plsc.store_scatter(ids_l, [slot], ids, mask=keep)
                plsc.store_scatter(pos_l, [slot], pos, mask=keep)
                return gv + jnp.sum(m.astype(jnp.int32))
            return lax.fori_loop(0, PIECE // L, vreg, g)
        return lax.fori_loop(0, B // PIECE, piece, jnp.int32(0))

    def extract(tbl, c):
        v = tbl[pl.ds(pl.multiple_of((c >> 4) * L, L), L)]
        return jnp.sum(jnp.where(_iota() == (c & (L - 1)), v, 0))

    def do_round(r, total0):
        skip = r * CAP
        total = lax.cond(r == 0,
                         lambda _: total_scan(skip),
                         lambda t: lax.cond(t > skip,
                                            lambda tt: total_scan(skip),
                                            lambda tt: tt,
                                            t),
                         total0)
        n = jnp.clip(total - skip, 0, CAP)
        nv = (n + L - 1) // L

        @pl.when(n > 0)
        def _():
            for b in range(256 // L):
                hist[pl.ds(b * L, L)] = jnp.zeros((L,), jnp.int32)

            def histp(v, _):
                ids = ids_l[pl.ds(v * L, L)]
                lanes = _iota() + v * L
                valid = lanes < n
                ch = jnp.where(valid, (ids - lo) >> 7, 255)
                plsc.addupdate_scatter(hist, [ch],
                                       jnp.ones((L,), jnp.int32),
                                       mask=valid)
                return 0
            lax.fori_loop(0, nv, histp, 0)

            carry = jnp.int32(0)
            for b in range(256 // L):
                h = hist[pl.ds(b * L, L)]
                excl = carry + lax.cumsum(h) - h
                starts[pl.ds(b * L, L)] = excl
                cursor[pl.ds(b * L, L)] = excl
                carry = carry + jnp.sum(h)

            def place(v, _):
                ids = ids_l[pl.ds(v * L, L)]
                pos = pos_l[pl.ds(v * L, L)]
                lanes = _iota() + v * L
                valid = lanes < n
                ch = jnp.where(valid, (ids - lo) >> 7, 255)
                pval = (ch << 21) | ((ids & (BLK - 1)) << 14) | pos
                chs, pvs = plsc.sort_key_val(ch, pval)
                vs = chs < 255
                rank = plsc.scan_count(chs)[0] - 1
                base = plsc.load_gather(cursor, [chs])
                plsc.store_scatter(pval_s, [base + rank], pvs, mask=vs)
                plsc.addupdate_scatter(cursor, [chs],
                                       jnp.ones((L,), jnp.int32),
                                       mask=vs)
                return 0
            lax.fori_loop(0, nv, place, 0)

            for s in range(NSLOT):
                fire(jnp.int32(s), s)

            def wave(wv, sn_w):
                c0 = wv * NSLOT
                for s in range(NSLOT):
                    drain(c0 + s, s)
                st = extract(starts, c0)
                en = extract(starts, c0 + NSLOT)
                cnt = en - st

                def itemg(g, sn_g):
                    addr = st + g * L + _iota()
                    m = addr < en
                    pv = plsc.load_gather(pval_s, [addr], mask=m)
                    slotv = (pv >> 21) - c0
                    loc = (pv >> 14) & (BLK - 1)
                    pos = pv & ((1 << 14) - 1)
                    slot = sn_g + lax.cumsum(m.astype(jnp.int32)) - 1
                    for j0 in range(0, D, 8):
                        vals = [plsc.load_gather(
                            win, [slotv, jnp.full((L,), j, jnp.int32), loc],
                            mask=m) for j in range(j0, j0 + 8)]
                        for j in range(j0, j0 + 8):
                            plsc.store_scatter(
                                stage, [slot, jnp.full((L,), j, jnp.int32)],
                                vals[j - j0], mask=m)
                    plsc.store_scatter(
                        spos, [jnp.zeros((L,), jnp.int32), slot],
                        pos, mask=m)
                    sn2 = sn_g + jnp.sum(m.astype(jnp.int32))

                    def doflush(x):
                        flush()
                        return jnp.int32(0)

                    return lax.cond(sn2 >= FT, doflush, lambda x: x, sn2)

                sn_w = lax.fori_loop(0, (cnt + L - 1) // L, itemg, sn_w)
                for s in range(NSLOT):
                    fire(c0 + NSLOT + s, s)
                return sn_w

            snf = lax.fori_loop(0, (BPT + NSLOT - 1) // NSLOT, wave,
                                jnp.int32(0))

            @pl.when(snf > 0)
            def _():
                flush()

        return total

        return 0

    lax.fori_loop(0, ROUNDS, do_round, jnp.int32(0))


def _sc_call(idx, ffeat, stats, tt, ttail):
    mesh = plsc.VectorSubcoreMesh(core_axis_name="c", subcore_axis_name="s")
    run = functools.partial(
        pl.kernel,
        mesh=mesh,
        compiler_params=pltpu.CompilerParams(use_tc_tiling_on_sc=True,
                                             needs_layout_passes=False),
        out_type=jax.ShapeDtypeStruct((OUTR, 128), jnp.float32),
        scratch_types=[
            pltpu.VMEM((CAP,), jnp.int32),
            pltpu.VMEM((CAP,), jnp.int32),
            pltpu.VMEM((CAP,), jnp.int32),
            pltpu.VMEM((256,), jnp.int32),
            pltpu.VMEM((256,), jnp.int32),
            pltpu.VMEM((256,), jnp.int32),
            pltpu.VMEM((NSLOT, D, BLK), jnp.float32),
            pltpu.VMEM((SROWS, 128), jnp.float32),
            pltpu.VMEM((1, SROWS), jnp.int32),
            pltpu.VMEM((SROWS, 128), jnp.float32),
            pltpu.VMEM((PIECE,), jnp.int32),
            pltpu.VMEM((L,), jnp.float32),
            pltpu.SemaphoreType.DMA,
            pltpu.SemaphoreType.DMA,
            pltpu.SemaphoreType.DMA,
            pltpu.SemaphoreType.DMA,
            pltpu.SemaphoreType.DMA,
            pltpu.SemaphoreType.DMA,
            pltpu.SemaphoreType.DMA,
            pltpu.SemaphoreType.DMA,
        ],
    )(_body)
    return run(idx, ffeat, stats, tt, ttail)


def kernel(visitorid, user_number_of_views, user_number_of_addtocart,
           user_number_of_purchases, number_of_unique_items,
           table, norm_mean, norm_var):
    idx = visitorid.astype(jnp.int32)
    inv_std = lax.rsqrt(norm_var.astype(jnp.float32) + 1e-7)
    stats = jnp.concatenate(
        [jnp.zeros((1,), jnp.float32), norm_mean.astype(jnp.float32),
         inv_std, jnp.zeros((L - 9,), jnp.float32)])
    feats = jnp.stack(
        [user_number_of_views, user_number_of_addtocart,
         user_number_of_purchases, number_of_unique_items], axis=1)
    ffeat = jnp.zeros((OUTR, 128), jnp.float32).at[:B, :4].set(feats)
    tt = table.T
    ttail = jnp.zeros((D, 128), jnp.float32).at[:, :V - TAIL].set(
        table[TAIL:].T)
    out = _sc_call(idx, ffeat, stats, tt, ttail)
    return out[:B, :DOUT]
